# baseline (speedup 1.0000x reference)
import functools

import jax
import jax.numpy as jnp
from jax import lax
from jax.experimental import pallas as pl
from jax.experimental.pallas import tpu as pltpu
from jax.experimental.pallas import tpu_sc as plsc

NC = 2
NS = 16
NW = NC * NS
CHUNK = 128


@functools.cache
def _build(n_chunks: int, dim: int):
    mesh = plsc.VectorSubcoreMesh(core_axis_name="c", subcore_axis_name="s")
    b_per_w = n_chunks * CHUNK

    @functools.partial(
        pl.kernel,
        out_type=jax.ShapeDtypeStruct((NW * b_per_w, dim), jnp.float32),
        mesh=mesh,
        scratch_types=[
            pltpu.VMEM((n_chunks, CHUNK), jnp.int32),
            pltpu.VMEM((CHUNK, dim), jnp.float32),
            pltpu.SemaphoreType.DMA,
        ],
        compiler_params=pltpu.CompilerParams(use_tc_tiling_on_sc=False),
    )
    def embed(idx_hbm, table_hbm, out_hbm, idx_v, rows_v, gsem):
        wid = lax.axis_index("s") * NC + lax.axis_index("c")
        base = wid * b_per_w
        pltpu.sync_copy(idx_hbm.at[wid], idx_v)

        def step(j, carry):
            pltpu.async_copy(table_hbm.at[idx_v.at[j]], rows_v, gsem).wait()
            pltpu.sync_copy(rows_v, out_hbm.at[pl.ds(base + j * CHUNK, CHUNK)])
            return carry

        lax.fori_loop(0, n_chunks, step, 0)

    return embed


def kernel(x, input_embedding):
    b_total = x.size
    dim = input_embedding.shape[1]
    n_chunks = b_total // (NW * CHUNK)
    xf = x.reshape(NW, n_chunks, CHUNK).astype(jnp.int32)
    out = _build(n_chunks, dim)(xf, input_embedding)
    return out.reshape(x.shape + (dim,))


# speedup vs baseline: 1.0442x; 1.0442x over previous
import functools

import jax
import jax.numpy as jnp
from jax import lax
from jax.experimental import pallas as pl
from jax.experimental.pallas import tpu as pltpu
from jax.experimental.pallas import tpu_sc as plsc

NC = 2
NS = 16
NW = NC * NS
CHUNK = 128
NBUF = 6
LOOKAHEAD = 4


@functools.cache
def _build(n_chunks: int, dim: int):
    mesh = plsc.VectorSubcoreMesh(core_axis_name="c", subcore_axis_name="s")
    b_per_w = n_chunks * CHUNK

    @functools.partial(
        pl.kernel,
        out_type=jax.ShapeDtypeStruct((NW * b_per_w, dim), jnp.float32),
        mesh=mesh,
        scratch_types=[
            pltpu.VMEM((n_chunks, CHUNK), jnp.int32),
            pltpu.VMEM((NBUF, CHUNK, dim), jnp.float32),
            pltpu.SemaphoreType.DMA((NBUF,)),
            pltpu.SemaphoreType.DMA((NBUF,)),
        ],
        compiler_params=pltpu.CompilerParams(use_tc_tiling_on_sc=False),
    )
    def embed(idx_hbm, table_hbm, out_hbm, idx_v, rows, gsem, ssem):
        wid = lax.axis_index("s") * NC + lax.axis_index("c")
        base = wid * b_per_w
        pltpu.sync_copy(idx_hbm.at[wid], idx_v)

        def gather_start(c, b):
            pltpu.async_copy(table_hbm.at[idx_v.at[c]], rows.at[b], gsem.at[b])

        def gather_wait(c, b):
            pltpu.make_async_copy(
                table_hbm.at[idx_v.at[c]], rows.at[b], gsem.at[b]).wait()

        def store_start(c, b):
            pltpu.async_copy(
                rows.at[b], out_hbm.at[pl.ds(base + c * CHUNK, CHUNK)],
                ssem.at[b])

        def store_wait(b):
            pltpu.make_async_copy(
                rows.at[b], out_hbm.at[pl.ds(base, CHUNK)], ssem.at[b]).wait()

        for c in range(LOOKAHEAD):
            gather_start(c, c % NBUF)

        def step(j, carry):
            c_pre = j + LOOKAHEAD

            @pl.when(c_pre < n_chunks)
            def _():
                b_pre = lax.rem(c_pre, NBUF)

                @pl.when(c_pre >= NBUF)
                def _():
                    store_wait(b_pre)

                gather_start(c_pre, b_pre)

            b = lax.rem(j, NBUF)
            gather_wait(j, b)
            store_start(j, b)
            return carry

        lax.fori_loop(0, n_chunks, step, 0)
        for b in range(NBUF):
            store_wait(b)

    return embed


def kernel(x, input_embedding):
    b_total = x.size
    dim = input_embedding.shape[1]
    n_chunks = b_total // (NW * CHUNK)
    xf = x.reshape(NW, n_chunks, CHUNK).astype(jnp.int32)
    out = _build(n_chunks, dim)(xf, input_embedding)
    return out.reshape(x.shape + (dim,))


# speedup vs baseline: 1.0445x; 1.0003x over previous
import functools

import jax
import jax.numpy as jnp
from jax import lax
from jax.experimental import pallas as pl
from jax.experimental.pallas import tpu as pltpu
from jax.experimental.pallas import tpu_sc as plsc

NC = 2
NS = 16
NW = NC * NS
GCH = 128
NBUF = 4
LOOKAHEAD = 2


@functools.cache
def _build(batch: int, seq: int, dim: int):
    mesh = plsc.VectorSubcoreMesh(core_axis_name="c", subcore_axis_name="s")
    rows_per_w = batch // NW
    splits = []
    off = 0
    while off < seq:
        splits.append((off, min(GCH, seq - off)))
        off += GCH

    @functools.partial(
        pl.kernel,
        out_type=jax.ShapeDtypeStruct((batch, seq, dim), jnp.float32),
        mesh=mesh,
        scratch_types=[
            pltpu.VMEM((rows_per_w, seq), jnp.int32),
            pltpu.VMEM((NBUF, seq, dim), jnp.float32),
            pltpu.SemaphoreType.DMA((NBUF,)),
            pltpu.SemaphoreType.DMA((NBUF,)),
        ],
        compiler_params=pltpu.CompilerParams(use_tc_tiling_on_sc=False),
    )
    def embed(x_hbm, table_hbm, out_hbm, idx_v, rows, gsem, ssem):
        wid = lax.axis_index("s") * NC + lax.axis_index("c")
        row0 = wid * rows_per_w
        pltpu.sync_copy(x_hbm.at[pl.ds(row0, rows_per_w)], idx_v)

        def gather_start(r, b):
            for off, n in splits:
                pltpu.async_copy(
                    table_hbm.at[idx_v.at[r, pl.ds(off, n)]],
                    rows.at[b, pl.ds(off, n)], gsem.at[b])

        def gather_wait(r, b):
            for off, n in splits:
                pltpu.make_async_copy(
                    table_hbm.at[idx_v.at[r, pl.ds(off, n)]],
                    rows.at[b, pl.ds(off, n)], gsem.at[b]).wait()

        def store_start(r, b):
            pltpu.async_copy(rows.at[b], out_hbm.at[row0 + r], ssem.at[b])

        def store_wait(b):
            pltpu.make_async_copy(rows.at[b], out_hbm.at[row0], ssem.at[b]).wait()

        for r in range(LOOKAHEAD):
            gather_start(r, r % NBUF)

        def step(r, carry):
            r_pre = r + LOOKAHEAD

            @pl.when(r_pre < rows_per_w)
            def _():
                b_pre = lax.rem(r_pre, NBUF)

                @pl.when(r_pre >= NBUF)
                def _():
                    store_wait(b_pre)

                gather_start(r_pre, b_pre)

            b = lax.rem(r, NBUF)
            gather_wait(r, b)
            store_start(r, b)
            return carry

        lax.fori_loop(0, rows_per_w, step, 0)
        for b in range(min(NBUF, rows_per_w)):
            store_wait(b)

    return embed


def kernel(x, input_embedding):
    batch, seq = x.shape
    dim = input_embedding.shape[1]
    return _build(batch, seq, dim)(x, input_embedding)
